# baseline (speedup 1.0000x reference)
import functools
import math

import jax
import jax.numpy as jnp
from jax import lax
from jax.experimental import pallas as pl
from jax.experimental.pallas import tpu as pltpu
from jax.experimental.pallas import tpu_sc as plsc

_VOCAB = 100000
_HID = 2048
_HEAD_DIM = 128
_BASE = 10000.0
_B, _S = 2, 2048

_NTOK = _B * _S
_NW = 32
_BPW = _NTOK // _NW
_CH = 16
_NCH = _BPW // _CH
_NBUF = 3
_WPB = _S // _BPW


@functools.partial(
    pl.kernel,
    out_type=jax.ShapeDtypeStruct((_NTOK, _HID), jnp.float32),
    mesh=plsc.VectorSubcoreMesh(core_axis_name="c", subcore_axis_name="s"),
    scratch_types=[
        pltpu.VMEM((_BPW,), jnp.int32),
        pltpu.VMEM((_CH, _HID), jnp.float32),
        pltpu.VMEM((_CH, _HID), jnp.float32),
        pltpu.VMEM((_CH, _HID), jnp.float32),
        pltpu.SemaphoreType.DMA,
        pltpu.SemaphoreType.DMA,
        pltpu.SemaphoreType.DMA,
        pltpu.SemaphoreType.DMA,
        pltpu.SemaphoreType.DMA,
        pltpu.SemaphoreType.DMA,
    ],
)
def _emb_gather(ids_hbm, w_hbm, out_hbm, idx_v,
                buf0, buf1, buf2, g0, g1, g2, o0, o1, o2):
    wid = lax.axis_index("s") * 2 + lax.axis_index("c")
    pltpu.sync_copy(
        ids_hbm.at[wid // _WPB, pl.ds((wid % _WPB) * _BPW, _BPW)], idx_v)

    bufs = (buf0, buf1, buf2)
    gsems = (g0, g1, g2)
    osems = (o0, o1, o2)

    def gather(k):
        b = k % _NBUF
        return pltpu.make_async_copy(
            w_hbm.at[idx_v.at[pl.ds(k * _CH, _CH)]], bufs[b], gsems[b])

    def out_copy(k):
        b = k % _NBUF
        return pltpu.make_async_copy(
            bufs[b], out_hbm.at[pl.ds(wid * _BPW + k * _CH, _CH)], osems[b])

    gather(0).start()
    gather(1).start()
    for k in range(_NCH):
        gather(k).wait()
        out_copy(k).start()
        nxt = k + _NBUF - 1
        if nxt < _NCH:
            if nxt >= _NBUF:
                out_copy(nxt - _NBUF).wait()
            gather(nxt).start()
    for k in range(_NCH - _NBUF, _NCH):
        out_copy(k).wait()


def _rot_body(cos_ref, sin_ref):
    shape = (1, 1, _S, _HEAD_DIM)
    pos = lax.broadcasted_iota(jnp.int32, shape, 2).astype(jnp.float32)
    col = lax.broadcasted_iota(jnp.int32, shape, 3)
    half = _HEAD_DIM // 2
    j = jnp.where(col < half, col, col - half).astype(jnp.float32)
    inv_freq = jnp.exp(j * (-math.log(_BASE) / half))
    freqs = pos * inv_freq
    cos_ref[...] = jnp.cos(freqs)
    sin_ref[...] = jnp.sin(freqs)


_rot = pl.pallas_call(
    _rot_body,
    grid=(_B,),
    out_shape=[jax.ShapeDtypeStruct((_B, 1, _S, _HEAD_DIM), jnp.float32)] * 2,
    out_specs=[pl.BlockSpec((1, 1, _S, _HEAD_DIM), lambda b: (b, 0, 0, 0))] * 2,
)


def kernel(input_ids, W):
    flat = _emb_gather(input_ids, W)
    hidden_states = flat.reshape(_B, _S, _HID)
    cos_g, sin_g = _rot()
    return (hidden_states, cos_g, sin_g)


# speedup vs baseline: 1.0112x; 1.0112x over previous
import functools
import math

import jax
import jax.numpy as jnp
from jax import lax
from jax.experimental import pallas as pl
from jax.experimental.pallas import tpu as pltpu
from jax.experimental.pallas import tpu_sc as plsc

_VOCAB = 100000
_HID = 2048
_HEAD_DIM = 128
_BASE = 10000.0
_B, _S = 2, 2048

_NTOK = _B * _S
_NW = 32
_BPW = _NTOK // _NW
_CH = 16
_NCH = _BPW // _CH
_NBUF = 3
_WPB = _S // _BPW


@functools.partial(
    pl.kernel,
    out_type=jax.ShapeDtypeStruct((_NTOK, _HID), jnp.float32),
    mesh=plsc.VectorSubcoreMesh(core_axis_name="c", subcore_axis_name="s"),
    scratch_types=[
        pltpu.VMEM((_BPW,), jnp.int32),
        pltpu.VMEM((_CH, _HID), jnp.float32),
        pltpu.VMEM((_CH, _HID), jnp.float32),
        pltpu.VMEM((_CH, _HID), jnp.float32),
        pltpu.SemaphoreType.DMA,
        pltpu.SemaphoreType.DMA,
        pltpu.SemaphoreType.DMA,
        pltpu.SemaphoreType.DMA,
        pltpu.SemaphoreType.DMA,
        pltpu.SemaphoreType.DMA,
    ],
)
def _emb_gather(ids_hbm, w_hbm, out_hbm, idx_v,
                buf0, buf1, buf2, g0, g1, g2, o0, o1, o2):
    wid = lax.axis_index("s") * 2 + lax.axis_index("c")
    pltpu.sync_copy(
        ids_hbm.at[wid // _WPB, pl.ds((wid % _WPB) * _BPW, _BPW)], idx_v)

    bufs = (buf0, buf1, buf2)
    gsems = (g0, g1, g2)
    osems = (o0, o1, o2)

    def gather(k):
        b = k % _NBUF
        return pltpu.make_async_copy(
            w_hbm.at[idx_v.at[pl.ds(k * _CH, _CH)]], bufs[b], gsems[b])

    def out_copy(k):
        b = k % _NBUF
        return pltpu.make_async_copy(
            bufs[b], out_hbm.at[pl.ds(wid * _BPW + k * _CH, _CH)], osems[b])

    gather(0).start()
    gather(1).start()
    for k in range(_NCH):
        gather(k).wait()
        out_copy(k).start()
        nxt = k + _NBUF - 1
        if nxt < _NCH:
            if nxt >= _NBUF:
                out_copy(nxt - _NBUF).wait()
            gather(nxt).start()
    for k in range(_NCH - _NBUF, _NCH):
        out_copy(k).wait()


def _rot_body(cos_ref, sin_ref):
    half = _HEAD_DIM // 2
    shape = (1, 1, _S, half)
    pos = lax.broadcasted_iota(jnp.int32, shape, 2).astype(jnp.float32)
    j = lax.broadcasted_iota(jnp.int32, shape, 3).astype(jnp.float32)
    inv_freq = jnp.exp(j * (-math.log(_BASE) / half))
    freqs = pos * inv_freq
    c = jnp.cos(freqs)
    s = jnp.sin(freqs)
    for b in range(_B):
        for h in range(2):
            cos_ref[b, :, :, pl.ds(h * half, half)] = c[0]
            sin_ref[b, :, :, pl.ds(h * half, half)] = s[0]


_rot = pl.pallas_call(
    _rot_body,
    out_shape=[jax.ShapeDtypeStruct((_B, 1, _S, _HEAD_DIM), jnp.float32)] * 2,
)


def kernel(input_ids, W):
    flat = _emb_gather(input_ids, W)
    hidden_states = flat.reshape(_B, _S, _HID)
    cos_g, sin_g = _rot()
    return (hidden_states, cos_g, sin_g)


# speedup vs baseline: 1.0124x; 1.0012x over previous
import functools
import math

import jax
import jax.numpy as jnp
from jax import lax
from jax.experimental import pallas as pl
from jax.experimental.pallas import tpu as pltpu
from jax.experimental.pallas import tpu_sc as plsc

_VOCAB = 100000
_HID = 2048
_HEAD_DIM = 128
_BASE = 10000.0
_B, _S = 2, 2048

_NTOK = _B * _S
_NW = 32
_BPW = _NTOK // _NW
_CH = 16
_NCH = _BPW // _CH
_NBUF = 3
_WPB = _S // _BPW


@functools.partial(
    pl.kernel,
    out_type=jax.ShapeDtypeStruct((_NTOK, _HID), jnp.float32),
    mesh=plsc.VectorSubcoreMesh(core_axis_name="c", subcore_axis_name="s"),
    scratch_types=[
        pltpu.VMEM((_BPW,), jnp.int32),
        pltpu.VMEM((_NBUF, _CH, _HID), jnp.float32),
        pltpu.SemaphoreType.DMA((_NBUF,)),
        pltpu.SemaphoreType.DMA((_NBUF,)),
    ],
)
def _emb_gather(ids_hbm, w_hbm, out_hbm, idx_v, bufs, gsem, osem):
    wid = lax.axis_index("s") * 2 + lax.axis_index("c")
    pltpu.sync_copy(
        ids_hbm.at[wid // _WPB, pl.ds((wid % _WPB) * _BPW, _BPW)], idx_v)

    def gather(k):
        b = k % _NBUF
        return pltpu.make_async_copy(
            w_hbm.at[idx_v.at[pl.ds(k * _CH, _CH)]], bufs.at[b], gsem.at[b])

    def out_copy(k):
        b = k % _NBUF
        return pltpu.make_async_copy(
            bufs.at[b], out_hbm.at[pl.ds(wid * _BPW + k * _CH, _CH)],
            osem.at[b])

    gather(0).start()
    gather(1).start()

    def step(k, carry):
        gather(k).wait()
        out_copy(k).start()

        @pl.when(k + 2 < _NCH)
        def _():
            @pl.when(k >= 1)
            def _():
                out_copy(k - 1).wait()
            gather(k + 2).start()

        return carry

    lax.fori_loop(0, _NCH, step, 0, unroll=False)

    def drain(k, carry):
        out_copy(k).wait()
        return carry

    lax.fori_loop(_NCH - _NBUF, _NCH, drain, 0, unroll=False)


def _rot_body(cos_ref, sin_ref):
    half = _HEAD_DIM // 2
    shape = (1, 1, _S, half)
    pos = lax.broadcasted_iota(jnp.int32, shape, 2).astype(jnp.float32)
    j = lax.broadcasted_iota(jnp.int32, shape, 3).astype(jnp.float32)
    inv_freq = jnp.exp(j * (-math.log(_BASE) / half))
    freqs = pos * inv_freq
    c = jnp.cos(freqs)
    s = jnp.sin(freqs)
    for b in range(_B):
        for h in range(2):
            cos_ref[b, :, :, pl.ds(h * half, half)] = c[0]
            sin_ref[b, :, :, pl.ds(h * half, half)] = s[0]


_rot = pl.pallas_call(
    _rot_body,
    out_shape=[jax.ShapeDtypeStruct((_B, 1, _S, _HEAD_DIM), jnp.float32)] * 2,
)


def kernel(input_ids, W):
    flat = _emb_gather(input_ids, W)
    hidden_states = flat.reshape(_B, _S, _HID)
    cos_g, sin_g = _rot()
    return (hidden_states, cos_g, sin_g)
